# baseline (speedup 1.0000x reference)
import functools
import math

import jax
import jax.numpy as jnp
from jax import lax
from jax.experimental import pallas as pl
from jax.experimental.pallas import tpu as pltpu

_MASK_VALUE = -0.7 * float(jnp.finfo(jnp.float32).max)
_N_HEADS = 16
_N_KV_HEADS = 4


def _qkv_kernel(x_ref, wq_ref, wk_ref, wv_ref, bq_ref, bk_ref, bv_ref,
                q_ref, k_ref, v_ref, xb_ref, *, n_rep, hd, n_kv, q_scale):
    p = pl.program_id(1)
    nq_tiles = pl.num_programs(1) - 2

    @pl.when(p == 0)
    def _cast_x():
        xb_ref[...] = x_ref[0].astype(jnp.bfloat16)

    x = xb_ref[...]

    @pl.when(p < nq_tiles)
    def _q():
        acc = lax.dot_general(x, wq_ref[...].astype(jnp.bfloat16),
                              (((1,), (1,)), ((), ())),
                              preferred_element_type=jnp.float32)
        y = (acc + bq_ref[...]) * q_scale
        for r in range(n_rep):
            q_ref[0, r] = y[:, r * hd:(r + 1) * hd].astype(q_ref.dtype)

    @pl.when(p == nq_tiles)
    def _k():
        acc = lax.dot_general(x, wk_ref[...].astype(jnp.bfloat16),
                              (((1,), (1,)), ((), ())),
                              preferred_element_type=jnp.float32)
        y = acc + bk_ref[...]
        for g in range(n_kv):
            k_ref[0, g] = y[:, g * hd:(g + 1) * hd].astype(k_ref.dtype)

    @pl.when(p == nq_tiles + 1)
    def _v():
        acc = lax.dot_general(x, wv_ref[...].astype(jnp.bfloat16),
                              (((1,), (1,)), ((), ())),
                              preferred_element_type=jnp.float32)
        y = acc + bv_ref[...]
        for g in range(n_kv):
            v_ref[0, g] = y[:, g * hd:(g + 1) * hd].astype(v_ref.dtype)


def _qkv_proj(xb, wq, wk, wv, bq, bk, bv, *, n_heads, n_kv_heads, hd):
    B, S, D = xb.shape
    n_rep = n_heads // n_kv_heads
    gw = n_rep * hd
    kvw = n_kv_heads * hd
    grid = (B, n_kv_heads + 2)
    last_q = n_kv_heads - 1
    q_scale = math.log2(math.e) / math.sqrt(hd)
    cost = pl.CostEstimate(
        flops=2 * B * S * D * (n_heads * hd + 2 * kvw), transcendentals=0,
        bytes_accessed=2 * (B * S * D + D * (n_heads * hd + 2 * kvw)
                            + B * S * (n_heads * hd + 2 * kvw)))
    return pl.pallas_call(
        functools.partial(_qkv_kernel, n_rep=n_rep, hd=hd, n_kv=n_kv_heads,
                          q_scale=q_scale),
        out_shape=(
            jax.ShapeDtypeStruct((B, n_heads, S, hd), jnp.bfloat16),
            jax.ShapeDtypeStruct((B, n_kv_heads, S, hd), jnp.bfloat16),
            jax.ShapeDtypeStruct((B, n_kv_heads, S, hd), jnp.bfloat16),
        ),
        grid_spec=pltpu.PrefetchScalarGridSpec(
            num_scalar_prefetch=0,
            grid=grid,
            in_specs=[
                pl.BlockSpec((1, S, D), lambda b, p: (b, 0, 0)),
                pl.BlockSpec((gw, D), lambda b, p: (jnp.minimum(p, last_q), 0)),
                pl.BlockSpec((kvw, D), lambda b, p: (0, 0)),
                pl.BlockSpec((kvw, D), lambda b, p: (0, 0)),
                pl.BlockSpec((1, gw), lambda b, p: (0, jnp.minimum(p, last_q))),
                pl.BlockSpec((1, kvw), lambda b, p: (0, 0)),
                pl.BlockSpec((1, kvw), lambda b, p: (0, 0)),
            ],
            out_specs=(
                pl.BlockSpec((1, n_rep, S, hd),
                             lambda b, p: (b, jnp.minimum(p, last_q), 0, 0)),
                pl.BlockSpec((1, n_kv_heads, S, hd), lambda b, p: (b, 0, 0, 0)),
                pl.BlockSpec((1, n_kv_heads, S, hd), lambda b, p: (b, 0, 0, 0)),
            ),
            scratch_shapes=[pltpu.VMEM((S, D), jnp.bfloat16)],
        ),
        compiler_params=pltpu.CompilerParams(
            dimension_semantics=("parallel", "arbitrary"),
            vmem_limit_bytes=48 * 1024 * 1024,
        ),
        cost_estimate=cost,
    )(xb, wq, wk, wv, bq, bk, bv)


def _flash_kernel(q_ref, k_ref, v_ref, o_ref, *, tq, tk, n_rep):
    qi = pl.program_id(2)
    hd = q_ref.shape[-1]

    qs = [q_ref[0, r] for r in range(n_rep)]

    def tile(ki_start, masked, carry):
        k = k_ref[0, 0, pl.ds(ki_start, tk), :]
        v = v_ref[0, 0, pl.ds(ki_start, tk), :]
        new = []
        for r in range(n_rep):
            m_prev, l_prev, acc_prev = carry[r]
            s = lax.dot_general(qs[r], k, (((1,), (1,)), ((), ())),
                                preferred_element_type=jnp.float32)
            if masked:
                cmask = (lax.broadcasted_iota(jnp.int32, (tq, tk), 0)
                         >= lax.broadcasted_iota(jnp.int32, (tq, tk), 1))
                s = jnp.where(cmask, s, _MASK_VALUE)
            m_new = jnp.maximum(m_prev, jnp.max(s, axis=-1, keepdims=True))
            alpha = jnp.exp2(m_prev - m_new)
            p = jnp.exp2(s - m_new)
            l_new = alpha * l_prev + jnp.sum(p, axis=-1, keepdims=True)
            pv = lax.dot_general(p.astype(v.dtype), v, (((1,), (0,)), ((), ())),
                                 preferred_element_type=jnp.float32)
            new.append((m_new, l_new, alpha * acc_prev + pv))
        return tuple(new)

    init = tuple((jnp.full((tq, 1), _MASK_VALUE, jnp.float32),
                  jnp.zeros((tq, 1), jnp.float32),
                  jnp.zeros((tq, hd), jnp.float32)) for _ in range(n_rep))

    def body(ki, carry):
        return tile(ki * tk, False, carry)

    carry = lax.fori_loop(0, qi, body, init)
    carry = tile(qi * tk, True, carry)

    for r in range(n_rep):
        _, l, acc = carry[r]
        inv_l = pl.reciprocal(l, approx=False)
        o_ref[0, :, r * hd:(r + 1) * hd] = (acc * inv_l).astype(o_ref.dtype)


def _flash_attention(q, k, v, *, n_rep, tq=128, tk=128):
    B, H, S, hd = q.shape
    n_kv = H // n_rep
    assert tq == tk and S % tq == 0
    nq = S // tq
    gw = n_rep * hd

    cost = pl.CostEstimate(
        flops=2 * B * H * S * S * hd,
        transcendentals=B * H * S * S // 2,
        bytes_accessed=2 * (2 * B * H * S * hd + 2 * B * n_kv * S * hd))

    return pl.pallas_call(
        functools.partial(_flash_kernel, tq=tq, tk=tk, n_rep=n_rep),
        out_shape=jax.ShapeDtypeStruct((B, S, H * hd), jnp.bfloat16),
        grid_spec=pltpu.PrefetchScalarGridSpec(
            num_scalar_prefetch=0,
            grid=(B, n_kv, nq),
            in_specs=[
                pl.BlockSpec((1, n_rep, tq, hd), lambda b, g, i: (b, g, i, 0)),
                pl.BlockSpec((1, 1, S, hd), lambda b, g, i: (b, g, 0, 0)),
                pl.BlockSpec((1, 1, S, hd), lambda b, g, i: (b, g, 0, 0)),
            ],
            out_specs=pl.BlockSpec((1, tq, gw), lambda b, g, i: (b, i, g)),
        ),
        compiler_params=pltpu.CompilerParams(
            dimension_semantics=("parallel", "parallel", "parallel"),
            vmem_limit_bytes=48 * 1024 * 1024,
        ),
        cost_estimate=cost,
    )(q, k, v)


def _proj_kernel(x_ref, w_ref, b_ref, o_ref, wb_ref):
    b = pl.program_id(1)

    @pl.when(b == 0)
    def _cast_w():
        wb_ref[...] = w_ref[...].astype(jnp.bfloat16)

    o_ref[0] = (lax.dot_general(x_ref[0], wb_ref[...], (((1,), (1,)), ((), ())),
                                preferred_element_type=jnp.float32)
                + b_ref[...]).astype(o_ref.dtype)


def _out_proj(attn, wo, bo, *, tn=1024):
    B, S, D = attn.shape
    N = wo.shape[0]
    grid = (N // tn, B)
    cost = pl.CostEstimate(flops=2 * B * S * D * N, transcendentals=0,
                           bytes_accessed=2 * B * S * D + 4 * (D * N + B * S * N))
    return pl.pallas_call(
        _proj_kernel,
        out_shape=jax.ShapeDtypeStruct((B, S, N), jnp.float32),
        grid_spec=pltpu.PrefetchScalarGridSpec(
            num_scalar_prefetch=0,
            grid=grid,
            in_specs=[
                pl.BlockSpec((1, S, D), lambda j, b: (b, 0, 0)),
                pl.BlockSpec((tn, D), lambda j, b: (j, 0)),
                pl.BlockSpec((1, tn), lambda j, b: (0, j)),
            ],
            out_specs=pl.BlockSpec((1, S, tn), lambda j, b: (b, 0, j)),
            scratch_shapes=[pltpu.VMEM((tn, D), jnp.bfloat16)],
        ),
        compiler_params=pltpu.CompilerParams(
            dimension_semantics=("parallel", "arbitrary"),
            vmem_limit_bytes=48 * 1024 * 1024,
        ),
        cost_estimate=cost,
    )(attn, wo, bo)


def kernel(x, wq, bq, wk, bk, wv, bv, wo, bo):
    B, S, D = x.shape
    H, Hkv = _N_HEADS, _N_KV_HEADS
    hd = D // H
    n_rep = H // Hkv

    q, k, v = _qkv_proj(x, wq, wk, wv,
                        bq.reshape(1, -1), bk.reshape(1, -1), bv.reshape(1, -1),
                        n_heads=H, n_kv_heads=Hkv, hd=hd)
    attn = _flash_attention(q, k, v, n_rep=n_rep)
    out = _out_proj(attn, wo, bo.reshape(1, -1))
    return out.astype(x.dtype)


# speedup vs baseline: 2.7390x; 2.7390x over previous
import functools
import math

import jax
import jax.numpy as jnp
from jax import lax
from jax.experimental import pallas as pl
from jax.experimental.pallas import tpu as pltpu

_MASK_VALUE = -0.7 * float(jnp.finfo(jnp.float32).max)
_N_HEADS = 16
_N_KV_HEADS = 4


def _qkv_kernel(x_ref, wq_ref, wk_ref, wv_ref, bq_ref, bk_ref, bv_ref,
                q_ref, k_ref, v_ref, xb_ref, *, n_rep, hd, n_kv, q_scale):
    p = pl.program_id(1)
    nq_tiles = pl.num_programs(1) - 2

    @pl.when(p == 0)
    def _cast_x():
        xb_ref[...] = x_ref[0].astype(jnp.bfloat16)

    x = xb_ref[...]

    @pl.when(p < nq_tiles)
    def _q():
        acc = lax.dot_general(x, wq_ref[...].astype(jnp.bfloat16),
                              (((1,), (1,)), ((), ())),
                              preferred_element_type=jnp.float32)
        y = (acc + bq_ref[...]) * q_scale
        for r in range(n_rep):
            q_ref[0, r] = y[:, r * hd:(r + 1) * hd].astype(q_ref.dtype)

    @pl.when(p == nq_tiles)
    def _k():
        acc = lax.dot_general(x, wk_ref[...].astype(jnp.bfloat16),
                              (((1,), (1,)), ((), ())),
                              preferred_element_type=jnp.float32)
        y = acc + bk_ref[...]
        for g in range(n_kv):
            k_ref[0, g] = y[:, g * hd:(g + 1) * hd].astype(k_ref.dtype)

    @pl.when(p == nq_tiles + 1)
    def _v():
        acc = lax.dot_general(x, wv_ref[...].astype(jnp.bfloat16),
                              (((1,), (1,)), ((), ())),
                              preferred_element_type=jnp.float32)
        y = acc + bv_ref[...]
        for g in range(n_kv):
            v_ref[0, g] = y[:, g * hd:(g + 1) * hd].astype(v_ref.dtype)


def _qkv_proj(xb, wq, wk, wv, bq, bk, bv, *, n_heads, n_kv_heads, hd):
    B, S, D = xb.shape
    n_rep = n_heads // n_kv_heads
    gw = n_rep * hd
    kvw = n_kv_heads * hd
    grid = (B, n_kv_heads + 2)
    last_q = n_kv_heads - 1
    q_scale = math.log2(math.e) / math.sqrt(hd)
    cost = pl.CostEstimate(
        flops=2 * B * S * D * (n_heads * hd + 2 * kvw), transcendentals=0,
        bytes_accessed=2 * (B * S * D + D * (n_heads * hd + 2 * kvw)
                            + B * S * (n_heads * hd + 2 * kvw)))
    return pl.pallas_call(
        functools.partial(_qkv_kernel, n_rep=n_rep, hd=hd, n_kv=n_kv_heads,
                          q_scale=q_scale),
        out_shape=(
            jax.ShapeDtypeStruct((B, n_heads, S, hd), jnp.bfloat16),
            jax.ShapeDtypeStruct((B, n_kv_heads, S, hd), jnp.bfloat16),
            jax.ShapeDtypeStruct((B, n_kv_heads, S, hd), jnp.bfloat16),
        ),
        grid_spec=pltpu.PrefetchScalarGridSpec(
            num_scalar_prefetch=0,
            grid=grid,
            in_specs=[
                pl.BlockSpec((1, S, D), lambda b, p: (b, 0, 0)),
                pl.BlockSpec((gw, D), lambda b, p: (jnp.minimum(p, last_q), 0)),
                pl.BlockSpec((kvw, D), lambda b, p: (0, 0)),
                pl.BlockSpec((kvw, D), lambda b, p: (0, 0)),
                pl.BlockSpec((1, gw), lambda b, p: (0, jnp.minimum(p, last_q))),
                pl.BlockSpec((1, kvw), lambda b, p: (0, 0)),
                pl.BlockSpec((1, kvw), lambda b, p: (0, 0)),
            ],
            out_specs=(
                pl.BlockSpec((1, n_rep, S, hd),
                             lambda b, p: (b, jnp.minimum(p, last_q), 0, 0)),
                pl.BlockSpec((1, n_kv_heads, S, hd), lambda b, p: (b, 0, 0, 0)),
                pl.BlockSpec((1, n_kv_heads, S, hd), lambda b, p: (b, 0, 0, 0)),
            ),
            scratch_shapes=[pltpu.VMEM((S, D), jnp.bfloat16)],
        ),
        compiler_params=pltpu.CompilerParams(
            dimension_semantics=("parallel", "arbitrary"),
            vmem_limit_bytes=48 * 1024 * 1024,
        ),
        cost_estimate=cost,
    )(xb, wq, wk, wv, bq, bk, bv)


def _flash_kernel(q_ref, k_ref, v_ref, o_ref, acc_ref, *, tq, tk, n_rep):
    qi = pl.program_id(2)
    hd = q_ref.shape[-1]

    qs = [q_ref[0, r] for r in range(n_rep)]

    def tile(ki_start, masked, carry):
        k = k_ref[0, 0, pl.ds(ki_start, tk), :]
        v = v_ref[0, 0, pl.ds(ki_start, tk), :]
        new = []
        for r in range(n_rep):
            m_prev, l_prev = carry[r]
            s = lax.dot_general(qs[r], k, (((1,), (1,)), ((), ())),
                                preferred_element_type=jnp.float32)
            if masked:
                cmask = (lax.broadcasted_iota(jnp.int32, (tq, tk), 0)
                         >= lax.broadcasted_iota(jnp.int32, (tq, tk), 1))
                s = jnp.where(cmask, s, _MASK_VALUE)
            m_new = jnp.maximum(m_prev, jnp.max(s, axis=-1, keepdims=True))
            alpha = jnp.exp2(m_prev - m_new)
            p = jnp.exp2(s - m_new)
            l_new = alpha * l_prev + jnp.sum(p, axis=-1, keepdims=True)
            pv = lax.dot_general(p.astype(v.dtype), v, (((1,), (0,)), ((), ())),
                                 preferred_element_type=jnp.float32)
            acc_ref[r] = alpha * acc_ref[r] + pv
            new.append((m_new, l_new))
        return tuple(new)

    for r in range(n_rep):
        acc_ref[r] = jnp.zeros((tq, hd), jnp.float32)

    init = tuple((jnp.full((tq, 1), _MASK_VALUE, jnp.float32),
                  jnp.zeros((tq, 1), jnp.float32)) for _ in range(n_rep))

    def body(ki, carry):
        return tile(ki * tk, False, carry)

    carry = lax.fori_loop(0, qi, body, init)
    carry = tile(qi * tk, True, carry)

    for r in range(n_rep):
        _, l = carry[r]
        inv_l = pl.reciprocal(l, approx=False)
        o_ref[0, :, r * hd:(r + 1) * hd] = (acc_ref[r] * inv_l).astype(o_ref.dtype)


def _flash_attention(q, k, v, *, n_rep, tq=256, tk=256):
    B, H, S, hd = q.shape
    n_kv = H // n_rep
    assert tq == tk and S % tq == 0
    nq = S // tq
    gw = n_rep * hd

    cost = pl.CostEstimate(
        flops=2 * B * H * S * S * hd,
        transcendentals=B * H * S * S // 2,
        bytes_accessed=2 * (2 * B * H * S * hd + 2 * B * n_kv * S * hd))

    return pl.pallas_call(
        functools.partial(_flash_kernel, tq=tq, tk=tk, n_rep=n_rep),
        out_shape=jax.ShapeDtypeStruct((B, S, H * hd), jnp.bfloat16),
        grid_spec=pltpu.PrefetchScalarGridSpec(
            num_scalar_prefetch=0,
            grid=(B, n_kv, nq),
            in_specs=[
                pl.BlockSpec((1, n_rep, tq, hd), lambda b, g, i: (b, g, i, 0)),
                pl.BlockSpec((1, 1, S, hd), lambda b, g, i: (b, g, 0, 0)),
                pl.BlockSpec((1, 1, S, hd), lambda b, g, i: (b, g, 0, 0)),
            ],
            out_specs=pl.BlockSpec((1, tq, gw), lambda b, g, i: (b, i, g)),
            scratch_shapes=[pltpu.VMEM((n_rep, tq, hd), jnp.float32)],
        ),
        compiler_params=pltpu.CompilerParams(
            dimension_semantics=("parallel", "parallel", "parallel"),
            vmem_limit_bytes=48 * 1024 * 1024,
        ),
        cost_estimate=cost,
    )(q, k, v)


def _proj_kernel(x_ref, w_ref, b_ref, o_ref, wb_ref):
    b = pl.program_id(1)

    @pl.when(b == 0)
    def _cast_w():
        wb_ref[...] = w_ref[...].astype(jnp.bfloat16)

    o_ref[0] = (lax.dot_general(x_ref[0], wb_ref[...], (((1,), (1,)), ((), ())),
                                preferred_element_type=jnp.float32)
                + b_ref[...]).astype(o_ref.dtype)


def _out_proj(attn, wo, bo, *, tn=1024):
    B, S, D = attn.shape
    N = wo.shape[0]
    grid = (N // tn, B)
    cost = pl.CostEstimate(flops=2 * B * S * D * N, transcendentals=0,
                           bytes_accessed=2 * B * S * D + 4 * (D * N + B * S * N))
    return pl.pallas_call(
        _proj_kernel,
        out_shape=jax.ShapeDtypeStruct((B, S, N), jnp.float32),
        grid_spec=pltpu.PrefetchScalarGridSpec(
            num_scalar_prefetch=0,
            grid=grid,
            in_specs=[
                pl.BlockSpec((1, S, D), lambda j, b: (b, 0, 0)),
                pl.BlockSpec((tn, D), lambda j, b: (j, 0)),
                pl.BlockSpec((1, tn), lambda j, b: (0, j)),
            ],
            out_specs=pl.BlockSpec((1, S, tn), lambda j, b: (b, 0, j)),
            scratch_shapes=[pltpu.VMEM((tn, D), jnp.bfloat16)],
        ),
        compiler_params=pltpu.CompilerParams(
            dimension_semantics=("parallel", "arbitrary"),
            vmem_limit_bytes=48 * 1024 * 1024,
        ),
        cost_estimate=cost,
    )(attn, wo, bo)


def kernel(x, wq, bq, wk, bk, wv, bv, wo, bo):
    B, S, D = x.shape
    H, Hkv = _N_HEADS, _N_KV_HEADS
    hd = D // H
    n_rep = H // Hkv

    q, k, v = _qkv_proj(x, wq, wk, wv,
                        bq.reshape(1, -1), bk.reshape(1, -1), bv.reshape(1, -1),
                        n_heads=H, n_kv_heads=Hkv, hd=hd)
    attn = _flash_attention(q, k, v, n_rep=n_rep)
    out = _out_proj(attn, wo, bo.reshape(1, -1))
    return out.astype(x.dtype)
